# Initial kernel scaffold; baseline (speedup 1.0000x reference)
import jax
import jax.numpy as jnp
from jax.experimental import pallas as pl


def kernel(pred_boxes, anchors, all_targets, sizes):
    raise NotImplementedError("write your pallas kernel here")


# speedup vs baseline: 1.0634x; 1.0634x over previous
import functools

import jax
import jax.numpy as jnp
from jax import lax
from jax.experimental import pallas as pl
from jax.experimental.pallas import tpu as pltpu
from jax.experimental.pallas import tpu_sc as plsc

_NC = 2
_NS = 16
_NW = _NC * _NS
_Q = 8000
_CH = _Q // 16
_BS = 8
_TPW = 16
_MT = 4


def _build_sc_kernel():
  mesh = plsc.VectorSubcoreMesh(core_axis_name="c", subcore_axis_name="s")
  out_type = (
      jax.ShapeDtypeStruct((_NW * 64,), jnp.int32),
      jax.ShapeDtypeStruct((_NW * 64,), jnp.int32),
  )
  scratch = [
      pltpu.VMEM((4, _Q), jnp.float32),
      pltpu.VMEM((4, _TPW), jnp.float32),
      pltpu.VMEM((64,), jnp.int32),
      pltpu.VMEM((64,), jnp.int32),
  ]

  @functools.partial(
      pl.kernel, out_type=out_type, mesh=mesh, scratch_types=scratch,
      compiler_params=pltpu.CompilerParams(needs_layout_passes=False))
  def sc_matcher(boxes_hbm, tgts_hbm, out_i, out_j, comp_v, tgt_v,
                 acc_i, acc_j):
    wid = lax.axis_index("s") * _NC + lax.axis_index("c")
    img = wid // 4
    arr = (wid % 4) // 2
    half = wid % 2

    pltpu.sync_copy(boxes_hbm.at[img * 2 + arr], comp_v)
    pltpu.sync_copy(tgts_hbm.at[img * 2 + half], tgt_v)

    def conv(ch, carry):
      s = pl.ds(ch * 16, 16)
      x0 = comp_v[0, s]
      y0 = comp_v[1, s]
      x1 = comp_v[2, s]
      y1 = comp_v[3, s]
      comp_v[0, s] = x0 + x1
      comp_v[1, s] = y0 + y1
      comp_v[2, s] = (x1 - x0) * 2.0
      comp_v[3, s] = (y1 - y0) * 2.0
      return carry

    lax.fori_loop(0, _CH, conv, 0)

    tx0 = tgt_v[0, :]
    ty0 = tgt_v[1, :]
    tx1 = tgt_v[2, :]
    ty1 = tgt_v[3, :]
    tsxv = tx0 + tx1
    tsyv = ty0 + ty1
    twv = (tx1 - tx0) * 2.0
    thv = (ty1 - ty0) * 2.0

    inf = jnp.float32(jnp.inf)
    lane = lax.iota(jnp.int32, 16)

    for t in range(_TPW):
      tsx = tsxv[t]
      tsy = tsyv[t]
      tw = twv[t]
      thh = thv[t]

      def chunk(ch, carry, tsx=tsx, tsy=tsy, tw=tw, thh=thh):
        bv, bi, thr = carry
        s = pl.ds(ch * 16, 16)
        c = (jnp.abs(comp_v[0, s] - tsx) + jnp.abs(comp_v[1, s] - tsy)
             + jnp.abs(comp_v[2, s] - tw) + jnp.abs(comp_v[3, s] - thh))

        def merge(args):
          bv, bi, _ = args
          idxv = ch * 16 + lane
          dv, di = plsc.sort_key_val(c, idxv, descending=True)
          keep = bv <= dv
          mv = jnp.where(keep, bv, dv)
          mi = jnp.where(keep, bi, di)
          nbv, nbi = plsc.sort_key_val(mv, mi)
          return nbv, nbi, nbv[_MT - 1]

        return lax.cond(jnp.any(c < thr), merge, lambda args: args,
                        (bv, bi, thr))

      bv0 = jnp.full((16,), inf, jnp.float32)
      bi0 = jnp.zeros((16,), jnp.int32)
      _, bi, _ = lax.fori_loop(0, _CH, chunk, (bv0, bi0, inf))
      plsc.store_scatter(acc_i, [lane * 16 + t], bi, mask=lane < _MT)

    jrow = lane + half * 16
    for r in range(_MT):
      acc_j[pl.ds(r * 16, 16)] = jrow

    off = wid * 64
    pltpu.sync_copy(acc_i, out_i.at[pl.ds(off, 64)])
    pltpu.sync_copy(acc_j, out_j.at[pl.ds(off, 64)])

  return sc_matcher


def kernel(pred_boxes, anchors, all_targets, sizes):
  bs, q = pred_boxes.shape[:2]
  szs = all_targets.shape[0] // bs

  boxes = jnp.stack([pred_boxes, anchors], axis=1)
  boxes = boxes.transpose(0, 1, 3, 2).reshape(bs * 2, 4, q)
  tsel = jnp.stack([
      lax.dynamic_slice_in_dim(all_targets, i * sizes, szs, axis=0)
      for i in range(bs)
  ])
  tgts = (tsel.transpose(0, 2, 1)
          .reshape(bs, 4, 2, szs // 2)
          .transpose(0, 2, 1, 3)
          .reshape(bs * 2, 4, szs // 2))

  oi, oj = _build_sc_kernel()(boxes, tgts)

  def assemble(flat):
    return (flat.reshape(bs, 2, 2, _MT, 16)
            .transpose(0, 3, 1, 2, 4)
            .reshape(bs, 2 * _MT * szs))

  return assemble(oi), assemble(oj)
